# baseline (speedup 1.0000x reference)
import functools

import jax
import jax.numpy as jnp
from jax.experimental import pallas as pl
from jax.experimental.pallas import tpu as pltpu

Q = 4096
K = 100000
D = 512
NUM_NEIGHBORS = 64

Q_TILE = 256
K_TILE = 1024
K_PAD = 100352


def _matmul_body(x_ref, kb_ref, out_ref):
    j = pl.program_id(1)
    scores = jax.lax.dot_general(
        x_ref[...], kb_ref[...],
        dimension_numbers=(((1,), (1,)), ((), ())),
        preferred_element_type=jnp.float32,
    )
    col = j * K_TILE + jax.lax.broadcasted_iota(jnp.int32, (Q_TILE, K_TILE), 1)
    scores = jnp.where(col < K, scores, -jnp.inf)
    out_ref[...] = scores


@functools.partial(jax.jit, static_argnames=())
def kernel(x, kb_embs):
    kb = jnp.pad(kb_embs, ((0, K_PAD - K), (0, 0)))
    scores = pl.pallas_call(
        _matmul_body,
        grid=(Q // Q_TILE, K_PAD // K_TILE),
        in_specs=[
            pl.BlockSpec((Q_TILE, D), lambda i, j: (i, 0)),
            pl.BlockSpec((K_TILE, D), lambda i, j: (j, 0)),
        ],
        out_specs=pl.BlockSpec((Q_TILE, K_TILE), lambda i, j: (i, j)),
        out_shape=jax.ShapeDtypeStruct((Q, K_PAD), jnp.float32),
    )(x, kb)
    _, top_idx = jax.lax.top_k(scores, NUM_NEIGHBORS)
    return top_idx


# speedup vs baseline: 15.3164x; 15.3164x over previous
import functools

import jax
import jax.numpy as jnp
from jax import lax
from jax.experimental import pallas as pl
from jax.experimental.pallas import tpu as pltpu
from jax.experimental.pallas import tpu_sc as plsc

Q = 4096
K = 100000
D = 512
NN = 64
RB = 1024
KT = 1024
CH = 128
KP = ((K + KT - 1) // KT) * KT
NC = KP // CH
CPT = KT // CH
B = 128
NCH = 80
NW = 32
RPW = Q // NW
RB2 = 256


def _mm_body(x_ref, kb_ref, scores_ref, m_ref):
    j = pl.program_id(1)
    s = lax.dot_general(
        x_ref[...], kb_ref[...],
        dimension_numbers=(((1,), (1,)), ((), ())),
        preferred_element_type=jnp.float32)
    col = j * KT + lax.broadcasted_iota(jnp.int32, (RB, KT), 1)
    s = jnp.where(col < K, s, -jnp.inf)
    scores_ref[...] = s
    m_ref[...] = jnp.max(s.reshape(RB, CPT, CH), axis=-1)[None]


def _tau_body(m_ref, tau_ref):
    def body(t, w):
        mx = jnp.max(w, axis=1, keepdims=True)
        return jnp.where(w == mx, -jnp.inf, w)

    w = lax.fori_loop(0, NN - 1, body, m_ref[...])
    tau_ref[...] = jnp.max(w, axis=1, keepdims=True)


_mesh = plsc.VectorSubcoreMesh(core_axis_name="c", subcore_axis_name="s")


@functools.partial(
    pl.kernel,
    out_type=[jax.ShapeDtypeStruct((Q, B), jnp.float32),
              jax.ShapeDtypeStruct((Q, B), jnp.int32)],
    mesh=_mesh,
    scratch_types=[
        pltpu.VMEM((RPW + 16,), jnp.float32),
        pltpu.VMEM((NC,), jnp.float32),
        pltpu.VMEM((NCH + 16,), jnp.int32),
        pltpu.VMEM((NCH, CH), jnp.float32),
        pltpu.VMEM((B,), jnp.float32),
        pltpu.VMEM((B,), jnp.int32),
        pltpu.VMEM((16,), jnp.float32),
        pltpu.VMEM((16,), jnp.int32),
        pltpu.VMEM((16,), jnp.int32),
        pltpu.SemaphoreType.DMA,
    ],
    compiler_params=pltpu.CompilerParams(needs_layout_passes=False),
)
def _sc_filter(scores_hbm, m_hbm, tau_hbm, vals_hbm, idx_hbm,
               tau_v, m_v, lid_v, gath_v, cv_v, ci_v,
               tau_s, off_s, cid_s, sem):
    wid = lax.axis_index("s") * _mesh.num_cores + lax.axis_index("c")
    r0 = wid * RPW
    pltpu.sync_copy(tau_hbm.at[pl.ds(r0, RPW)], tau_v.at[pl.ds(0, RPW)])

    zeros16i = jnp.zeros((16,), jnp.int32)
    iota16 = lax.iota(jnp.int32, 16)
    neg16 = jnp.full((16,), -jnp.inf, jnp.float32)
    padchunk16 = jnp.full((16,), (K // CH) + 1, jnp.int32)

    fifteen16 = jnp.full((16,), 15, jnp.int32)
    one16 = jnp.full((16,), 1, jnp.int32)

    _gd = lax.GatherDimensionNumbers(
        offset_dims=(), collapsed_slice_dims=(0,), start_index_map=(0,))

    def _bcast(vec, idx16):
        return lax.gather(vec, idx16[:, None], _gd, (1,),
                          mode=lax.GatherScatterMode.PROMISE_IN_BOUNDS)

    def lane0(vec):
        return _bcast(vec, zeros16i)

    def lanelast(vec):
        return _bcast(vec, fifteen16)

    def row_body(rr, _):
        r = r0 + rr
        pltpu.sync_copy(m_hbm.at[r], m_v)
        tau_s[...] = lane0(tau_v[pl.ds(rr, 16)])

        def reset_body(v, _):
            lid_v[pl.ds(v * 16, 16)] = padchunk16
            return 0

        lax.fori_loop(0, NCH // 16, reset_body, 0)

        off_s[...] = zeros16i
        cid_s[...] = iota16

        def scan_body(v, _):
            mv = m_v[pl.ds(v * 16, 16)]
            msk = mv >= tau_s[...]
            cs = plsc.cumsum(jnp.where(msk, one16, zeros16i))
            off_v = off_s[...]
            pos = jnp.minimum(off_v + cs - 1, NCH - 1)
            plsc.store_scatter(lid_v, [pos], cid_s[...], mask=msk)
            off_s[...] = off_v + lanelast(cs)
            cid_s[...] = cid_s[...] + 16
            return 0

        lax.fori_loop(0, NC // 16, scan_body, 0)

        pltpu.async_copy(
            scores_hbm.at[r].at[lid_v.at[pl.ds(0, NCH)]], gath_v, sem).wait()

        def cinit(v, _):
            cv_v[pl.ds(v * 16, 16)] = neg16
            ci_v[pl.ds(v * 16, 16)] = zeros16i
            return 0

        lax.fori_loop(0, B // 16, cinit, 0)

        off_s[...] = zeros16i

        def ext_body(k, _):
            lid16 = lane0(lid_v[pl.ds(k, 16)])
            for l in range(CH // 16):
                vv = gath_v[k, pl.ds(l * 16, 16)]
                msk = vv >= tau_s[...]
                cs = plsc.cumsum(jnp.where(msk, one16, zeros16i))
                off_v = off_s[...]
                pos = jnp.minimum(off_v + cs - 1, B - 1)
                gidx = lid16 * CH + (l * 16) + iota16
                plsc.store_scatter(cv_v, [pos], vv, mask=msk)
                plsc.store_scatter(ci_v, [pos], gidx, mask=msk)
                off_s[...] = off_v + lanelast(cs)
            return 0

        lax.fori_loop(0, NCH, ext_body, 0)

        pltpu.sync_copy(cv_v, vals_hbm.at[r])
        pltpu.sync_copy(ci_v, idx_hbm.at[r])
        return 0

    lax.fori_loop(0, RPW, row_body, 0)


def _topk_body(v_ref, i_ref, o_ref):
    v = v_ref[...]
    ind = i_ref[...]
    lane = lax.broadcasted_iota(jnp.int32, (RB2, NN), 1)
    BIG = jnp.int32(2**30)

    def body(t, carry):
        v, out = carry
        mx = jnp.max(v, axis=1, keepdims=True)
        cand = jnp.where(v == mx, ind, BIG)
        imin = jnp.min(cand, axis=1, keepdims=True)
        out = jnp.where(lane == t, imin, out)
        v = jnp.where((v == mx) & (ind == imin), -jnp.inf, v)
        return v, out

    _, out = lax.fori_loop(0, NN, body, (v, jnp.zeros((RB2, NN), jnp.int32)))
    o_ref[...] = out


@jax.jit
def kernel(x, kb_embs):
    kbp = jnp.pad(kb_embs, ((0, KP - K), (0, 0)))
    scores, m3 = pl.pallas_call(
        _mm_body,
        grid=(Q // RB, KP // KT),
        in_specs=[
            pl.BlockSpec((RB, D), lambda i, j: (i, 0)),
            pl.BlockSpec((KT, D), lambda i, j: (j, 0)),
        ],
        out_specs=[
            pl.BlockSpec((RB, KT), lambda i, j: (i, j)),
            pl.BlockSpec((1, RB, CPT), lambda i, j: (j, i, 0)),
        ],
        out_shape=[
            jax.ShapeDtypeStruct((Q, KP), jnp.float32),
            jax.ShapeDtypeStruct((KP // KT, Q, CPT), jnp.float32),
        ],
    )(x, kbp)

    m = m3.transpose(1, 0, 2).reshape(Q, NC)
    tau = pl.pallas_call(
        _tau_body,
        grid=(Q // RB2,),
        in_specs=[pl.BlockSpec((RB2, NC), lambda i: (i, 0))],
        out_specs=pl.BlockSpec((RB2, 1), lambda i: (i, 0)),
        out_shape=jax.ShapeDtypeStruct((Q, 1), jnp.float32),
    )(m)

    cv, ci = _sc_filter(scores.reshape(Q, NC, CH), m, tau.reshape(Q))

    return pl.pallas_call(
        _topk_body,
        grid=(Q // RB2,),
        in_specs=[
            pl.BlockSpec((RB2, B), lambda i: (i, 0)),
            pl.BlockSpec((RB2, B), lambda i: (i, 0)),
        ],
        out_specs=pl.BlockSpec((RB2, NN), lambda i: (i, 0)),
        out_shape=jax.ShapeDtypeStruct((Q, NN), jnp.int32),
    )(cv, ci)
